# baseline (speedup 1.0000x reference)
import jax
import jax.numpy as jnp
from jax.experimental import pallas as pl
from jax.experimental.pallas import tpu as pltpu

TOKENS = 32768
D = 1024
E = 8
BT = 2048
GRID = TOKENS // (2 * BT)


def _gating_kernel(xa_ref, xb_ref, w_ref, b_ref, o1_ref, o2_ref):
    w = w_ref[...]
    b = b_ref[...]
    ga = (
        jax.lax.dot_general(
            w, xa_ref[...], (((1,), (1,)), ((), ())),
            preferred_element_type=jnp.float32,
        )
        + b
    )
    gb = (
        jax.lax.dot_general(
            w, xb_ref[...], (((1,), (1,)), ((), ())),
            preferred_element_type=jnp.float32,
        )
        + b
    )
    o1_ref[:, :BT] = ga
    o1_ref[:, BT:] = gb
    o2_ref[:, :BT] = ga
    o2_ref[:, BT:] = gb


def kernel(x, W, b, train):
    b2 = b.reshape(E, 1)
    gt1, gt2 = pl.pallas_call(
        _gating_kernel,
        grid=(GRID,),
        in_specs=[
            pl.BlockSpec((BT, D), lambda i: (2 * i, 0)),
            pl.BlockSpec((BT, D), lambda i: (2 * i + 1, 0)),
            pl.BlockSpec((E, D), lambda i: (0, 0)),
            pl.BlockSpec((E, 1), lambda i: (0, 0)),
        ],
        out_specs=[
            pl.BlockSpec((E, 2 * BT), lambda i: (0, i)),
            pl.BlockSpec((E, 2 * BT), lambda i: (0, i)),
        ],
        out_shape=[
            jax.ShapeDtypeStruct((E, TOKENS), jnp.float32),
            jax.ShapeDtypeStruct((E, TOKENS), jnp.float32),
        ],
        compiler_params=pltpu.CompilerParams(
            dimension_semantics=("parallel",),
            vmem_limit_bytes=48 * 1024 * 1024,
        ),
    )(x, x, W, b2)
    return (gt1.T, gt2.T)


# speedup vs baseline: 1.0511x; 1.0511x over previous
import jax
import jax.numpy as jnp
from jax.experimental import pallas as pl
from jax.experimental.pallas import tpu as pltpu

TOKENS = 32768
D = 1024
E = 8
BT = 2048


def _gating_kernel(x_ref, w_ref, b_ref, o1_ref, o2_ref):
    g = (
        jax.lax.dot_general(
            w_ref[...],
            x_ref[...],
            (((1,), (1,)), ((), ())),
            preferred_element_type=jnp.float32,
        )
        + b_ref[...]
    )
    o1_ref[...] = g
    o2_ref[...] = g


def kernel(x, W, b, train):
    b2 = b.reshape(E, 1)
    gt1, gt2 = pl.pallas_call(
        _gating_kernel,
        grid=(TOKENS // BT,),
        in_specs=[
            pl.BlockSpec((BT, D), lambda i: (i, 0)),
            pl.BlockSpec((E, D), lambda i: (0, 0)),
            pl.BlockSpec((E, 1), lambda i: (0, 0)),
        ],
        out_specs=[
            pl.BlockSpec((E, BT), lambda i: (0, i)),
            pl.BlockSpec((E, BT), lambda i: (0, i)),
        ],
        out_shape=[
            jax.ShapeDtypeStruct((E, TOKENS), jnp.float32),
            jax.ShapeDtypeStruct((E, TOKENS), jnp.float32),
        ],
        compiler_params=pltpu.CompilerParams(
            dimension_semantics=("parallel",),
        ),
    )(x, W, b2)
    return (gt1.T, gt2.T)
